# baseline (speedup 1.0000x reference)
import functools

import jax
import jax.numpy as jnp
from jax import lax
from jax.experimental import pallas as pl
from jax.experimental.pallas import tpu as pltpu
from jax.experimental.pallas import tpu_sc as plsc

_NC = 2
_NS = 16
_L = 16
_NW = _NC * _NS
_B = 8 * 64
_BPW = _B // _NW
_D = 64

_mesh = plsc.VectorSubcoreMesh(core_axis_name="c", subcore_axis_name="s")


@functools.partial(
    pl.kernel,
    mesh=_mesh,
    out_type=jax.ShapeDtypeStruct((_B, _D), jnp.float32),
    compiler_params=pltpu.CompilerParams(use_tc_tiling_on_sc=False),
    scratch_types=[
        pltpu.VMEM((_L,), jnp.int32),
        pltpu.VMEM((_L,), jnp.int32),
        pltpu.VMEM((_BPW,), jnp.int32),
        pltpu.VMEM((_BPW, _D), jnp.float32),
        pltpu.SemaphoreType.DMA,
    ],
)
def _gather_sc(x_hbm, pack_hbm, out_hbm, i1_v, i2_v, idx_v, rows_v, sem):
    wid = lax.axis_index("s") * _NC + lax.axis_index("c")
    p = wid // 4
    jbase = (wid % 4) * _BPW
    pltpu.sync_copy(pack_hbm.at[p], i1_v)
    pltpu.sync_copy(pack_hbm.at[p + 8], i2_v)
    lane = lax.iota(jnp.int32, _L)
    idx_v[...] = (i1_v[...] * 64 + jbase + lane) * 128 + i2_v[...]
    pltpu.async_copy(x_hbm.at[idx_v], rows_v, sem).wait()
    pltpu.sync_copy(rows_v, out_hbm.at[pl.ds(wid * _BPW, _BPW)])


def kernel(x, index1, index2):
    xf = x.reshape(128 * 64 * 128, 64)
    pairs = jnp.concatenate(
        [index1.reshape(8).astype(jnp.int32), index2.reshape(8).astype(jnp.int32)]
    )
    pack = jnp.broadcast_to(pairs[:, None], (2 * 8, _L))
    return _gather_sc(xf, pack).reshape(4, 2, 64, 64)


# speedup vs baseline: 1.6452x; 1.6452x over previous
import functools

import jax
import jax.numpy as jnp
from jax import lax
from jax.experimental import pallas as pl
from jax.experimental.pallas import tpu as pltpu
from jax.experimental.pallas import tpu_sc as plsc

_NC = 2
_NS = 16
_L = 16
_NW = _NC * _NS
_B = 8 * 64
_BPW = _B // _NW

_mesh = plsc.VectorSubcoreMesh(core_axis_name="c", subcore_axis_name="s")


@functools.partial(
    pl.kernel,
    mesh=_mesh,
    out_type=jax.ShapeDtypeStruct((_B, 64), jnp.float32),
    scratch_types=[
        pltpu.VMEM((_L,), jnp.int32),
        pltpu.VMEM((_L,), jnp.int32),
        pltpu.VMEM((_BPW, 64), jnp.float32),
    ],
)
def _gather_sc(x_hbm, pack_hbm, out_hbm, i1_v, i2_v, rows_v):
    wid = lax.axis_index("s") * _NC + lax.axis_index("c")
    p = wid // 4
    jbase = (wid % 4) * _BPW
    pltpu.sync_copy(pack_hbm.at[p], i1_v)
    pltpu.sync_copy(pack_hbm.at[p + 8], i2_v)
    i1 = i1_v[...][0]
    i2 = i2_v[...][0]
    pltpu.sync_copy(x_hbm.at[i1, pl.ds(jbase, _BPW), i2, :], rows_v)
    pltpu.sync_copy(rows_v, out_hbm.at[pl.ds(wid * _BPW, _BPW), :])


def kernel(x, index1, index2):
    pairs = jnp.concatenate(
        [index1.reshape(8).astype(jnp.int32), index2.reshape(8).astype(jnp.int32)]
    )
    pack = jnp.broadcast_to(pairs[:, None], (2 * 8, _L))
    return _gather_sc(x, pack).reshape(4, 2, 64, 64)
